# Initial kernel scaffold; baseline (speedup 1.0000x reference)
import jax
import jax.numpy as jnp
from jax.experimental import pallas as pl


def kernel(h, edge_index, coord, edge_attr, Wq, bq, Wkv, bkv, W1, W2):
    raise NotImplementedError("write your pallas kernel here")


# speedup vs baseline: 9.0041x; 9.0041x over previous
import functools

import jax
import jax.numpy as jnp
from jax import lax
from jax.experimental import pallas as pl
from jax.experimental.pallas import tpu as pltpu
from jax.experimental.pallas import tpu_sc as plsc

N = 10000
E = 320000
D = 128
C = 3
ED = 16
H = 128

TW = 144
EB = 512
NBLK = E // EB
LAM = 8.0

NC = 2
NS = 16
NW = NC * NS
EPW = E // NW
EPS = E // NS
CH = 80

SL = 624
SLAST = N - SL * (NS - 1)


def _sc_mesh():
    return plsc.VectorSubcoreMesh(
        core_axis_name="c", subcore_axis_name="s",
        num_cores=NC, num_subcores=NS)


def _sc_gather(table, row, col):
    @functools.partial(
        pl.kernel, mesh=_sc_mesh(),
        out_type=(jax.ShapeDtypeStruct((E, TW), jnp.float32),
                  jax.ShapeDtypeStruct((E, TW), jnp.float32)),
        scratch_types=[
            pltpu.VMEM((CH,), jnp.int32),
            pltpu.VMEM((CH,), jnp.int32),
            pltpu.VMEM((CH, TW), jnp.float32),
            pltpu.VMEM((CH, TW), jnp.float32),
        ],
        compiler_params=pltpu.CompilerParams(use_tc_tiling_on_sc=False),
    )
    def gather_kernel(t_hbm, row_hbm, col_hbm, outr_hbm, outc_hbm,
                      idxr_v, idxc_v, bufr_v, bufc_v):
        wid = lax.axis_index("s") * NC + lax.axis_index("c")
        base = wid * EPW

        def body(i, carry):
            off = base + i * CH
            pltpu.sync_copy(row_hbm.at[pl.ds(off, CH)], idxr_v)
            pltpu.sync_copy(col_hbm.at[pl.ds(off, CH)], idxc_v)
            pltpu.sync_copy(t_hbm.at[idxr_v], bufr_v)
            pltpu.sync_copy(t_hbm.at[idxc_v], bufc_v)
            pltpu.sync_copy(bufr_v, outr_hbm.at[pl.ds(off, CH)])
            pltpu.sync_copy(bufc_v, outc_hbm.at[pl.ds(off, CH)])
            return carry

        lax.fori_loop(0, EPW // CH, body, 0)

    return gather_kernel(table, row, col)


def _sc_gather_s1(s1, row):
    @functools.partial(
        pl.kernel, mesh=_sc_mesh(),
        out_type=jax.ShapeDtypeStruct((E, 16), jnp.float32),
        scratch_types=[
            pltpu.VMEM((CH,), jnp.int32),
            pltpu.VMEM((CH, 16), jnp.float32),
        ],
        compiler_params=pltpu.CompilerParams(use_tc_tiling_on_sc=False),
    )
    def gather_kernel(s1_hbm, row_hbm, out_hbm, idx_v, buf_v):
        wid = lax.axis_index("s") * NC + lax.axis_index("c")
        base = wid * EPW

        def body(i, carry):
            off = base + i * CH
            pltpu.sync_copy(row_hbm.at[pl.ds(off, CH)], idx_v)
            pltpu.sync_copy(s1_hbm.at[idx_v], buf_v)
            pltpu.sync_copy(buf_v, out_hbm.at[pl.ds(off, CH)])
            return carry

        lax.fori_loop(0, EPW // CH, body, 0)

    return gather_kernel(s1, row)


def _sc_scatter_s1(w1e, row, zeros_n16):
    @functools.partial(
        pl.kernel, mesh=_sc_mesh(),
        out_type=jax.ShapeDtypeStruct((N, 16), jnp.float32),
        scratch_types=[
            pltpu.VMEM((CH,), jnp.int32),
            pltpu.VMEM((CH, 16), jnp.float32),
            pltpu.VMEM_SHARED((N, 16), jnp.float32),
        ],
        compiler_params=pltpu.CompilerParams(use_tc_tiling_on_sc=False),
    )
    def scatter_kernel(w_hbm, row_hbm, z_hbm, out_hbm, idx_v, buf_v, acc_sh):
        cid = lax.axis_index("c")
        sid = lax.axis_index("s")

        @pl.when(cid == 0)
        def _():
            @pl.when(sid < NS - 1)
            def _():
                pltpu.sync_copy(z_hbm.at[pl.ds(sid * SL, SL)],
                                acc_sh.at[pl.ds(sid * SL, SL)])

            @pl.when(sid == NS - 1)
            def _():
                pltpu.sync_copy(z_hbm.at[pl.ds((NS - 1) * SL, SLAST)],
                                acc_sh.at[pl.ds((NS - 1) * SL, SLAST)])

            plsc.subcore_barrier()
            base = sid * EPS

            def body(i, carry):
                off = base + i * CH
                pltpu.sync_copy(row_hbm.at[pl.ds(off, CH)], idx_v)
                pltpu.sync_copy(w_hbm.at[pl.ds(off, CH)], buf_v)
                pltpu.sync_copy(buf_v, acc_sh.at[idx_v], add=True)
                return carry

            lax.fori_loop(0, EPS // CH, body, 0)
            plsc.subcore_barrier()

            @pl.when(sid < NS - 1)
            def _():
                pltpu.sync_copy(acc_sh.at[pl.ds(sid * SL, SL)],
                                out_hbm.at[pl.ds(sid * SL, SL)])

            @pl.when(sid == NS - 1)
            def _():
                pltpu.sync_copy(acc_sh.at[pl.ds((NS - 1) * SL, SLAST)],
                                out_hbm.at[pl.ds((NS - 1) * SL, SLAST)])

    return scatter_kernel(w1e, row, zeros_n16)


def _sc_scatter(payload, row, zeros_nt):
    @functools.partial(
        pl.kernel, mesh=_sc_mesh(),
        out_type=jax.ShapeDtypeStruct((NC, N, TW), jnp.float32),
        scratch_types=[
            pltpu.VMEM((CH,), jnp.int32),
            pltpu.VMEM((CH, TW), jnp.float32),
            pltpu.VMEM_SHARED((N, TW), jnp.float32),
        ],
        compiler_params=pltpu.CompilerParams(use_tc_tiling_on_sc=False),
    )
    def scatter_kernel(pay_hbm, row_hbm, z_hbm, out_hbm, idx_v, buf_v, acc_sh):
        cid = lax.axis_index("c")
        sid = lax.axis_index("s")
        wid = sid * NC + cid
        base = wid * EPW

        @pl.when(sid < NS - 1)
        def _():
            pltpu.sync_copy(z_hbm.at[pl.ds(sid * SL, SL)],
                            acc_sh.at[pl.ds(sid * SL, SL)])

        @pl.when(sid == NS - 1)
        def _():
            pltpu.sync_copy(z_hbm.at[pl.ds((NS - 1) * SL, SLAST)],
                            acc_sh.at[pl.ds((NS - 1) * SL, SLAST)])

        plsc.subcore_barrier()

        def body(i, carry):
            off = base + i * CH
            pltpu.sync_copy(row_hbm.at[pl.ds(off, CH)], idx_v)
            pltpu.sync_copy(pay_hbm.at[pl.ds(off, CH)], buf_v)
            pltpu.sync_copy(buf_v, acc_sh.at[idx_v], add=True)
            return carry

        lax.fori_loop(0, EPW // CH, body, 0)
        plsc.subcore_barrier()

        @pl.when(sid < NS - 1)
        def _():
            pltpu.sync_copy(acc_sh.at[pl.ds(sid * SL, SL)],
                            out_hbm.at[cid, pl.ds(sid * SL, SL)])

        @pl.when(sid == NS - 1)
        def _():
            pltpu.sync_copy(acc_sh.at[pl.ds((NS - 1) * SL, SLAST)],
                            out_hbm.at[cid, pl.ds((NS - 1) * SL, SLAST)])

    return scatter_kernel(payload, row, zeros_nt)


def _radial16(cd):
    cols = []
    for i in range(C):
        a = cd[:, 3 * i:3 * i + 3]
        for j in range(C):
            b = cd[:, 3 * j:3 * j + 3]
            cols.append(jnp.sum(a * b, axis=1, keepdims=True))
    cols.append(jnp.zeros((cd.shape[0], 7), jnp.float32))
    return jnp.concatenate(cols, axis=1)


def _tc_alpha(hrT, hcT, ea, Wq, bq, Wk_r, Wk_h, Wk_e, bk):
    def body(hr_ref, hc_ref, ea_ref, wq_ref, bq_ref, wkr_ref, wkh_ref,
             wke_ref, bk_ref, alpha_ref, cd_ref, w1_ref):
        hr = hr_ref[:, :D]
        cr = hr_ref[:, D:D + 9]
        hc = hc_ref[:, :D]
        cc = hc_ref[:, D:D + 9]
        cd = cr - cc
        cd_ref[...] = jnp.concatenate(
            [cd, jnp.zeros((EB, 7), jnp.float32)], axis=1)
        rad = _radial16(cd)
        q = jnp.dot(hr, wq_ref[...], preferred_element_type=jnp.float32)
        q = q + bq_ref[...]
        k = (jnp.dot(rad, wkr_ref[...], preferred_element_type=jnp.float32)
             + jnp.dot(hc, wkh_ref[...], preferred_element_type=jnp.float32)
             + jnp.dot(ea_ref[...], wke_ref[...],
                       preferred_element_type=jnp.float32)
             + bk_ref[...])
        alpha = jnp.sum(q * k, axis=1, keepdims=True)
        alpha_ref[...] = alpha
        w1_ref[...] = jnp.concatenate(
            [jnp.exp(alpha * (1.0 / LAM)), jnp.zeros((EB, 15), jnp.float32)],
            axis=1)

    full = lambda s: pl.BlockSpec(s, lambda i: (0, 0))
    return pl.pallas_call(
        body,
        grid=(NBLK,),
        in_specs=[
            pl.BlockSpec((EB, TW), lambda i: (i, 0)),
            pl.BlockSpec((EB, TW), lambda i: (i, 0)),
            pl.BlockSpec((EB, ED), lambda i: (i, 0)),
            full((D, H)), full((1, H)), full((16, H)), full((D, H)),
            full((ED, H)), full((1, H)),
        ],
        out_specs=[
            pl.BlockSpec((EB, 1), lambda i: (i, 0)),
            pl.BlockSpec((EB, 16), lambda i: (i, 0)),
            pl.BlockSpec((EB, 16), lambda i: (i, 0)),
        ],
        out_shape=[
            jax.ShapeDtypeStruct((E, 1), jnp.float32),
            jax.ShapeDtypeStruct((E, 16), jnp.float32),
            jax.ShapeDtypeStruct((E, 16), jnp.float32),
        ],
        compiler_params=pltpu.CompilerParams(
            dimension_semantics=("parallel",)),
    )(hrT, hcT, ea, Wq, bq, Wk_r, Wk_h, Wk_e, bk)


def _tc_payload(hcT, cd16, ea, alpha, s1r, Wv_r, Wv_h, Wv_e, bv, W1, W2p):
    def body(hc_ref, cd_ref, ea_ref, al_ref, s1_ref, wvr_ref, wvh_ref,
             wve_ref, bv_ref, w1_ref, w2_ref, out_ref):
        hc = hc_ref[:, :D]
        cd = cd_ref[:, :9]
        rad = _radial16(cd)
        v = (jnp.dot(rad, wvr_ref[...], preferred_element_type=jnp.float32)
             + jnp.dot(hc, wvh_ref[...], preferred_element_type=jnp.float32)
             + jnp.dot(ea_ref[...], wve_ref[...],
                       preferred_element_type=jnp.float32)
             + bv_ref[...])
        ex = jnp.exp(al_ref[...] - LAM * jnp.log(s1_ref[:, 0:1]))
        u = jnp.dot(v, w1_ref[...], preferred_element_type=jnp.float32)
        u = u * jax.nn.sigmoid(u)
        cv = jnp.dot(u, w2_ref[...], preferred_element_type=jnp.float32)
        tcols = []
        for i in range(C):
            ci = cv[:, i:i + 1]
            for d in range(3):
                tcols.append(cd[:, 3 * i + d:3 * i + d + 1] * ci)
        trans = jnp.concatenate(tcols, axis=1)
        out_ref[...] = jnp.concatenate(
            [ex * v, ex * trans, ex, jnp.zeros((EB, 6), jnp.float32)],
            axis=1)

    full = lambda s: pl.BlockSpec(s, lambda i: (0, 0))
    return pl.pallas_call(
        body,
        grid=(NBLK,),
        in_specs=[
            pl.BlockSpec((EB, TW), lambda i: (i, 0)),
            pl.BlockSpec((EB, 16), lambda i: (i, 0)),
            pl.BlockSpec((EB, ED), lambda i: (i, 0)),
            pl.BlockSpec((EB, 1), lambda i: (i, 0)),
            pl.BlockSpec((EB, 16), lambda i: (i, 0)),
            full((16, H)), full((D, H)), full((ED, H)), full((1, H)),
            full((H, 2 * H)), full((2 * H, 16)),
        ],
        out_specs=pl.BlockSpec((EB, TW), lambda i: (i, 0)),
        out_shape=jax.ShapeDtypeStruct((E, TW), jnp.float32),
        compiler_params=pltpu.CompilerParams(
            dimension_semantics=("parallel",)),
    )(hcT, cd16, ea, alpha, s1r, Wv_r, Wv_h, Wv_e, bv, W1, W2p)


def _tc_finalize(partials, h, coord9):
    NB = 1000

    def body(p_ref, h_ref, c_ref, ho_ref, co_ref):
        u = p_ref[0] + p_ref[1]
        s = u[:, D + 9:D + 10]
        inv = jnp.where(s > 0, 1.0 / jnp.where(s > 0, s, 1.0), 0.0)
        ho_ref[...] = h_ref[...] + u[:, :D] * inv
        aggc = u[:, D:D + 9] * inv
        co_ref[...] = c_ref[...] + jnp.clip(aggc, -10.0, 10.0)

    return pl.pallas_call(
        body,
        grid=(N // NB,),
        in_specs=[
            pl.BlockSpec((NC, NB, TW), lambda i: (0, i, 0)),
            pl.BlockSpec((NB, D), lambda i: (i, 0)),
            pl.BlockSpec((NB, 9), lambda i: (i, 0)),
        ],
        out_specs=[
            pl.BlockSpec((NB, D), lambda i: (i, 0)),
            pl.BlockSpec((NB, 9), lambda i: (i, 0)),
        ],
        out_shape=[
            jax.ShapeDtypeStruct((N, D), jnp.float32),
            jax.ShapeDtypeStruct((N, 9), jnp.float32),
        ],
        compiler_params=pltpu.CompilerParams(
            dimension_semantics=("parallel",)),
    )(partials, h, coord9)


def kernel(h, edge_index, coord, edge_attr, Wq, bq, Wkv, bkv, W1, W2):
    row = edge_index[0]
    col = edge_index[1]
    coord9 = coord.reshape(N, C * 3)

    table = jnp.concatenate(
        [h, coord9, jnp.zeros((N, TW - D - C * 3), jnp.float32)], axis=1)

    Wk = Wkv[:, 0::2]
    Wv = Wkv[:, 1::2]
    bk = bkv[0::2].reshape(1, H)
    bv = bkv[1::2].reshape(1, H)
    Wk_r = jnp.concatenate([Wk[:C * C], jnp.zeros((7, H), jnp.float32)], 0)
    Wv_r = jnp.concatenate([Wv[:C * C], jnp.zeros((7, H), jnp.float32)], 0)
    Wk_h = Wk[C * C:C * C + D]
    Wv_h = Wv[C * C:C * C + D]
    Wk_e = Wk[C * C + D:]
    Wv_e = Wv[C * C + D:]
    W2p = jnp.concatenate([W2, jnp.zeros((2 * H, 16 - C), jnp.float32)], 1)
    bq2 = bq.reshape(1, H)

    hrT, hcT = _sc_gather(table, row, col)
    alpha, cd16, w1e = _tc_alpha(hrT, hcT, edge_attr,
                                 Wq, bq2, Wk_r, Wk_h, Wk_e, bk)
    s1 = _sc_scatter_s1(w1e, row, jnp.zeros((N, 16), jnp.float32))
    s1r = _sc_gather_s1(s1, row)
    payload = _tc_payload(hcT, cd16, edge_attr, alpha, s1r,
                          Wv_r, Wv_h, Wv_e, bv, W1, W2p)
    partials = _sc_scatter(payload, row, jnp.zeros((N, TW), jnp.float32))
    h_out, coord_out9 = _tc_finalize(partials, h, coord9)
    return (h_out, coord_out9.reshape(N, C, 3))


# speedup vs baseline: 10.3346x; 1.1478x over previous
import functools

import jax
import jax.numpy as jnp
from jax import lax
from jax.experimental import pallas as pl
from jax.experimental.pallas import tpu as pltpu
from jax.experimental.pallas import tpu_sc as plsc

N = 10000
E = 320000
D = 128
C = 3
ED = 16
H = 128

TW = 144
EB = 512
NBLK = E // EB
LAM = 8.0

NC = 2
NS = 16
NW = NC * NS
EPW = E // NW
EPS = E // NS
CH = 80

SL = 624
SLAST = N - SL * (NS - 1)


def _sc_mesh():
    return plsc.VectorSubcoreMesh(
        core_axis_name="c", subcore_axis_name="s",
        num_cores=NC, num_subcores=NS)


def _sc_gather(table, row, col):
    @functools.partial(
        pl.kernel, mesh=_sc_mesh(),
        out_type=(jax.ShapeDtypeStruct((E, TW), jnp.float32),
                  jax.ShapeDtypeStruct((E, TW), jnp.float32)),
        scratch_types=[
            pltpu.VMEM((EPW,), jnp.int32),
            pltpu.VMEM((EPW,), jnp.int32),
            pltpu.VMEM((CH, TW), jnp.float32),
            pltpu.VMEM((CH, TW), jnp.float32),
            pltpu.VMEM((CH, TW), jnp.float32),
            pltpu.VMEM((CH, TW), jnp.float32),
            pltpu.SemaphoreType.DMA,
            pltpu.SemaphoreType.DMA,
            pltpu.SemaphoreType.DMA,
        ],
        compiler_params=pltpu.CompilerParams(use_tc_tiling_on_sc=False),
    )
    def gather_kernel(t_hbm, row_hbm, col_hbm, outr_hbm, outc_hbm,
                      idxr_all, idxc_all, br0, bc0, br1, bc1,
                      semg0, semg1, semw):
        wid = lax.axis_index("s") * NC + lax.axis_index("c")
        base = wid * EPW
        pltpu.sync_copy(row_hbm.at[pl.ds(base, EPW)], idxr_all)
        pltpu.sync_copy(col_hbm.at[pl.ds(base, EPW)], idxc_all)

        def g_start(ch, br, bc, sem):
            s = pl.ds(ch * CH, CH)
            a = pltpu.async_copy(t_hbm.at[idxr_all.at[s]], br, sem)
            b = pltpu.async_copy(t_hbm.at[idxc_all.at[s]], bc, sem)
            return a, b

        def w_start(ch, br, bc, sem):
            off = base + ch * CH
            a = pltpu.async_copy(br, outr_hbm.at[pl.ds(off, CH)], sem)
            b = pltpu.async_copy(bc, outc_hbm.at[pl.ds(off, CH)], sem)
            return a, b

        def body(j, carry):
            c0 = 2 * j
            g0a, g0b = g_start(c0, br0, bc0, semg0)
            g1a, g1b = g_start(c0 + 1, br1, bc1, semg1)
            g0a.wait()
            g0b.wait()
            w0a, w0b = w_start(c0, br0, bc0, semw)
            g1a.wait()
            g1b.wait()
            w1a, w1b = w_start(c0 + 1, br1, bc1, semw)
            w0a.wait()
            w0b.wait()
            w1a.wait()
            w1b.wait()
            return carry

        npair = (EPW // CH) // 2
        lax.fori_loop(0, npair, body, 0)
        tc = EPW // CH - 1
        ga, gb = g_start(tc, br0, bc0, semg0)
        ga.wait()
        gb.wait()
        wa, wb = w_start(tc, br0, bc0, semw)
        wa.wait()
        wb.wait()

    return gather_kernel(table, row, col)


def _sc_gather_s1(s1, row):
    @functools.partial(
        pl.kernel, mesh=_sc_mesh(),
        out_type=jax.ShapeDtypeStruct((E, 16), jnp.float32),
        scratch_types=[
            pltpu.VMEM((EPW,), jnp.int32),
            pltpu.VMEM((CH, 16), jnp.float32),
            pltpu.VMEM((CH, 16), jnp.float32),
            pltpu.SemaphoreType.DMA,
            pltpu.SemaphoreType.DMA,
            pltpu.SemaphoreType.DMA,
        ],
        compiler_params=pltpu.CompilerParams(use_tc_tiling_on_sc=False),
    )
    def gather_kernel(s1_hbm, row_hbm, out_hbm, idx_all, b0, b1,
                      semg0, semg1, semw):
        wid = lax.axis_index("s") * NC + lax.axis_index("c")
        base = wid * EPW
        pltpu.sync_copy(row_hbm.at[pl.ds(base, EPW)], idx_all)

        def g_start(ch, b, sem):
            return pltpu.async_copy(
                s1_hbm.at[idx_all.at[pl.ds(ch * CH, CH)]], b, sem)

        def w_start(ch, b, sem):
            off = base + ch * CH
            return pltpu.async_copy(b, out_hbm.at[pl.ds(off, CH)], sem)

        def body(j, carry):
            c0 = 2 * j
            g0 = g_start(c0, b0, semg0)
            g1 = g_start(c0 + 1, b1, semg1)
            g0.wait()
            w0 = w_start(c0, b0, semw)
            g1.wait()
            w1 = w_start(c0 + 1, b1, semw)
            w0.wait()
            w1.wait()
            return carry

        lax.fori_loop(0, (EPW // CH) // 2, body, 0)
        tc = EPW // CH - 1
        g = g_start(tc, b0, semg0)
        g.wait()
        w = w_start(tc, b0, semw)
        w.wait()

    return gather_kernel(s1, row)


def _sc_scatter_s1(w1e, row, zeros_n16):
    @functools.partial(
        pl.kernel, mesh=_sc_mesh(),
        out_type=jax.ShapeDtypeStruct((N, 16), jnp.float32),
        scratch_types=[
            pltpu.VMEM((CH,), jnp.int32),
            pltpu.VMEM((CH,), jnp.int32),
            pltpu.VMEM((CH, 16), jnp.float32),
            pltpu.VMEM((CH, 16), jnp.float32),
            pltpu.VMEM_SHARED((N, 16), jnp.float32),
            pltpu.SemaphoreType.DMA,
            pltpu.SemaphoreType.DMA,
        ],
        compiler_params=pltpu.CompilerParams(use_tc_tiling_on_sc=False),
    )
    def scatter_kernel(w_hbm, row_hbm, z_hbm, out_hbm, idx0_v, idx1_v,
                       buf0_v, buf1_v, acc_sh, sem0, sem1):
        cid = lax.axis_index("c")
        sid = lax.axis_index("s")

        @pl.when(cid == 0)
        def _():
            @pl.when(sid < NS - 1)
            def _():
                pltpu.sync_copy(z_hbm.at[pl.ds(sid * SL, SL)],
                                acc_sh.at[pl.ds(sid * SL, SL)])

            @pl.when(sid == NS - 1)
            def _():
                pltpu.sync_copy(z_hbm.at[pl.ds((NS - 1) * SL, SLAST)],
                                acc_sh.at[pl.ds((NS - 1) * SL, SLAST)])

            plsc.subcore_barrier()
            base = sid * EPS

            def l_start(ch, idx_v, buf_v, sem):
                off = base + ch * CH
                a = pltpu.async_copy(row_hbm.at[pl.ds(off, CH)], idx_v, sem)
                b = pltpu.async_copy(w_hbm.at[pl.ds(off, CH)], buf_v, sem)
                return a, b

            def body(j, carry):
                c0 = 2 * j
                a0, b0 = l_start(c0, idx0_v, buf0_v, sem0)
                a1, b1 = l_start(c0 + 1, idx1_v, buf1_v, sem1)
                a0.wait()
                b0.wait()
                pltpu.sync_copy(buf0_v, acc_sh.at[idx0_v], add=True)
                a1.wait()
                b1.wait()
                pltpu.sync_copy(buf1_v, acc_sh.at[idx1_v], add=True)
                return carry

            lax.fori_loop(0, (EPS // CH) // 2, body, 0)
            plsc.subcore_barrier()

            @pl.when(sid < NS - 1)
            def _():
                pltpu.sync_copy(acc_sh.at[pl.ds(sid * SL, SL)],
                                out_hbm.at[pl.ds(sid * SL, SL)])

            @pl.when(sid == NS - 1)
            def _():
                pltpu.sync_copy(acc_sh.at[pl.ds((NS - 1) * SL, SLAST)],
                                out_hbm.at[pl.ds((NS - 1) * SL, SLAST)])

    return scatter_kernel(w1e, row, zeros_n16)


def _sc_scatter(payload, row, zeros_nt):
    @functools.partial(
        pl.kernel, mesh=_sc_mesh(),
        out_type=jax.ShapeDtypeStruct((NC, N, TW), jnp.float32),
        scratch_types=[
            pltpu.VMEM((CH,), jnp.int32),
            pltpu.VMEM((CH,), jnp.int32),
            pltpu.VMEM((CH, TW), jnp.float32),
            pltpu.VMEM((CH, TW), jnp.float32),
            pltpu.VMEM_SHARED((N, TW), jnp.float32),
            pltpu.SemaphoreType.DMA,
            pltpu.SemaphoreType.DMA,
        ],
        compiler_params=pltpu.CompilerParams(use_tc_tiling_on_sc=False),
    )
    def scatter_kernel(pay_hbm, row_hbm, z_hbm, out_hbm, idx0_v, idx1_v,
                       buf0_v, buf1_v, acc_sh, sem0, sem1):
        cid = lax.axis_index("c")
        sid = lax.axis_index("s")
        wid = sid * NC + cid
        base = wid * EPW

        @pl.when(sid < NS - 1)
        def _():
            pltpu.sync_copy(z_hbm.at[pl.ds(sid * SL, SL)],
                            acc_sh.at[pl.ds(sid * SL, SL)])

        @pl.when(sid == NS - 1)
        def _():
            pltpu.sync_copy(z_hbm.at[pl.ds((NS - 1) * SL, SLAST)],
                            acc_sh.at[pl.ds((NS - 1) * SL, SLAST)])

        plsc.subcore_barrier()

        def l_start(ch, idx_v, buf_v, sem):
            off = base + ch * CH
            a = pltpu.async_copy(row_hbm.at[pl.ds(off, CH)], idx_v, sem)
            b = pltpu.async_copy(pay_hbm.at[pl.ds(off, CH)], buf_v, sem)
            return a, b

        def body(j, carry):
            c0 = 2 * j
            a0, b0 = l_start(c0, idx0_v, buf0_v, sem0)
            a1, b1 = l_start(c0 + 1, idx1_v, buf1_v, sem1)
            a0.wait()
            b0.wait()
            pltpu.sync_copy(buf0_v, acc_sh.at[idx0_v], add=True)
            a1.wait()
            b1.wait()
            pltpu.sync_copy(buf1_v, acc_sh.at[idx1_v], add=True)
            return carry

        npair = (EPW // CH) // 2
        lax.fori_loop(0, npair, body, 0)
        ta, tb = l_start(EPW // CH - 1, idx0_v, buf0_v, sem0)
        ta.wait()
        tb.wait()
        pltpu.sync_copy(buf0_v, acc_sh.at[idx0_v], add=True)
        plsc.subcore_barrier()

        @pl.when(sid < NS - 1)
        def _():
            pltpu.sync_copy(acc_sh.at[pl.ds(sid * SL, SL)],
                            out_hbm.at[cid, pl.ds(sid * SL, SL)])

        @pl.when(sid == NS - 1)
        def _():
            pltpu.sync_copy(acc_sh.at[pl.ds((NS - 1) * SL, SLAST)],
                            out_hbm.at[cid, pl.ds((NS - 1) * SL, SLAST)])

    return scatter_kernel(payload, row, zeros_nt)


def _radial16(cd):
    cols = []
    for i in range(C):
        a = cd[:, 3 * i:3 * i + 3]
        for j in range(C):
            b = cd[:, 3 * j:3 * j + 3]
            cols.append(jnp.sum(a * b, axis=1, keepdims=True))
    cols.append(jnp.zeros((cd.shape[0], 7), jnp.float32))
    return jnp.concatenate(cols, axis=1)


def _tc_alpha(hrT, hcT, ea, Wq, bq, Wk_r, Wk_h, Wk_e, bk):
    def body(hr_ref, hc_ref, ea_ref, wq_ref, bq_ref, wkr_ref, wkh_ref,
             wke_ref, bk_ref, alpha_ref, cd_ref, w1_ref):
        hr = hr_ref[:, :D]
        cr = hr_ref[:, D:D + 9]
        hc = hc_ref[:, :D]
        cc = hc_ref[:, D:D + 9]
        cd = cr - cc
        cd_ref[...] = jnp.concatenate(
            [cd, jnp.zeros((EB, 7), jnp.float32)], axis=1)
        rad = _radial16(cd)
        q = jnp.dot(hr, wq_ref[...], preferred_element_type=jnp.float32)
        q = q + bq_ref[...]
        k = (jnp.dot(rad, wkr_ref[...], preferred_element_type=jnp.float32)
             + jnp.dot(hc, wkh_ref[...], preferred_element_type=jnp.float32)
             + jnp.dot(ea_ref[...], wke_ref[...],
                       preferred_element_type=jnp.float32)
             + bk_ref[...])
        alpha = jnp.sum(q * k, axis=1, keepdims=True)
        alpha_ref[...] = alpha
        w1_ref[...] = jnp.concatenate(
            [jnp.exp(alpha * (1.0 / LAM)), jnp.zeros((EB, 15), jnp.float32)],
            axis=1)

    full = lambda s: pl.BlockSpec(s, lambda i: (0, 0))
    return pl.pallas_call(
        body,
        grid=(NBLK,),
        in_specs=[
            pl.BlockSpec((EB, TW), lambda i: (i, 0)),
            pl.BlockSpec((EB, TW), lambda i: (i, 0)),
            pl.BlockSpec((EB, ED), lambda i: (i, 0)),
            full((D, H)), full((1, H)), full((16, H)), full((D, H)),
            full((ED, H)), full((1, H)),
        ],
        out_specs=[
            pl.BlockSpec((EB, 1), lambda i: (i, 0)),
            pl.BlockSpec((EB, 16), lambda i: (i, 0)),
            pl.BlockSpec((EB, 16), lambda i: (i, 0)),
        ],
        out_shape=[
            jax.ShapeDtypeStruct((E, 1), jnp.float32),
            jax.ShapeDtypeStruct((E, 16), jnp.float32),
            jax.ShapeDtypeStruct((E, 16), jnp.float32),
        ],
        compiler_params=pltpu.CompilerParams(
            dimension_semantics=("parallel",)),
    )(hrT, hcT, ea, Wq, bq, Wk_r, Wk_h, Wk_e, bk)


def _tc_payload(hcT, cd16, ea, alpha, s1r, Wv_r, Wv_h, Wv_e, bv, W1, W2p):
    def body(hc_ref, cd_ref, ea_ref, al_ref, s1_ref, wvr_ref, wvh_ref,
             wve_ref, bv_ref, w1_ref, w2_ref, out_ref):
        hc = hc_ref[:, :D]
        cd = cd_ref[:, :9]
        rad = _radial16(cd)
        v = (jnp.dot(rad, wvr_ref[...], preferred_element_type=jnp.float32)
             + jnp.dot(hc, wvh_ref[...], preferred_element_type=jnp.float32)
             + jnp.dot(ea_ref[...], wve_ref[...],
                       preferred_element_type=jnp.float32)
             + bv_ref[...])
        ex = jnp.exp(al_ref[...] - LAM * jnp.log(s1_ref[:, 0:1]))
        u = jnp.dot(v, w1_ref[...], preferred_element_type=jnp.float32)
        u = u * jax.nn.sigmoid(u)
        cv = jnp.dot(u, w2_ref[...], preferred_element_type=jnp.float32)
        tcols = []
        for i in range(C):
            ci = cv[:, i:i + 1]
            for d in range(3):
                tcols.append(cd[:, 3 * i + d:3 * i + d + 1] * ci)
        trans = jnp.concatenate(tcols, axis=1)
        out_ref[...] = jnp.concatenate(
            [ex * v, ex * trans, ex, jnp.zeros((EB, 6), jnp.float32)],
            axis=1)

    full = lambda s: pl.BlockSpec(s, lambda i: (0, 0))
    return pl.pallas_call(
        body,
        grid=(NBLK,),
        in_specs=[
            pl.BlockSpec((EB, TW), lambda i: (i, 0)),
            pl.BlockSpec((EB, 16), lambda i: (i, 0)),
            pl.BlockSpec((EB, ED), lambda i: (i, 0)),
            pl.BlockSpec((EB, 1), lambda i: (i, 0)),
            pl.BlockSpec((EB, 16), lambda i: (i, 0)),
            full((16, H)), full((D, H)), full((ED, H)), full((1, H)),
            full((H, 2 * H)), full((2 * H, 16)),
        ],
        out_specs=pl.BlockSpec((EB, TW), lambda i: (i, 0)),
        out_shape=jax.ShapeDtypeStruct((E, TW), jnp.float32),
        compiler_params=pltpu.CompilerParams(
            dimension_semantics=("parallel",)),
    )(hcT, cd16, ea, alpha, s1r, Wv_r, Wv_h, Wv_e, bv, W1, W2p)


def _tc_finalize(partials, h, coord9):
    NB = 1000

    def body(p_ref, h_ref, c_ref, ho_ref, co_ref):
        u = p_ref[0] + p_ref[1]
        s = u[:, D + 9:D + 10]
        inv = jnp.where(s > 0, 1.0 / jnp.where(s > 0, s, 1.0), 0.0)
        ho_ref[...] = h_ref[...] + u[:, :D] * inv
        aggc = u[:, D:D + 9] * inv
        co_ref[...] = c_ref[...] + jnp.clip(aggc, -10.0, 10.0)

    return pl.pallas_call(
        body,
        grid=(N // NB,),
        in_specs=[
            pl.BlockSpec((NC, NB, TW), lambda i: (0, i, 0)),
            pl.BlockSpec((NB, D), lambda i: (i, 0)),
            pl.BlockSpec((NB, 9), lambda i: (i, 0)),
        ],
        out_specs=[
            pl.BlockSpec((NB, D), lambda i: (i, 0)),
            pl.BlockSpec((NB, 9), lambda i: (i, 0)),
        ],
        out_shape=[
            jax.ShapeDtypeStruct((N, D), jnp.float32),
            jax.ShapeDtypeStruct((N, 9), jnp.float32),
        ],
        compiler_params=pltpu.CompilerParams(
            dimension_semantics=("parallel",)),
    )(partials, h, coord9)


def kernel(h, edge_index, coord, edge_attr, Wq, bq, Wkv, bkv, W1, W2):
    row = edge_index[0]
    col = edge_index[1]
    coord9 = coord.reshape(N, C * 3)

    table = jnp.concatenate(
        [h, coord9, jnp.zeros((N, TW - D - C * 3), jnp.float32)], axis=1)

    Wk = Wkv[:, 0::2]
    Wv = Wkv[:, 1::2]
    bk = bkv[0::2].reshape(1, H)
    bv = bkv[1::2].reshape(1, H)
    Wk_r = jnp.concatenate([Wk[:C * C], jnp.zeros((7, H), jnp.float32)], 0)
    Wv_r = jnp.concatenate([Wv[:C * C], jnp.zeros((7, H), jnp.float32)], 0)
    Wk_h = Wk[C * C:C * C + D]
    Wv_h = Wv[C * C:C * C + D]
    Wk_e = Wk[C * C + D:]
    Wv_e = Wv[C * C + D:]
    W2p = jnp.concatenate([W2, jnp.zeros((2 * H, 16 - C), jnp.float32)], 1)
    bq2 = bq.reshape(1, H)

    hrT, hcT = _sc_gather(table, row, col)
    alpha, cd16, w1e = _tc_alpha(hrT, hcT, edge_attr,
                                 Wq, bq2, Wk_r, Wk_h, Wk_e, bk)
    s1 = _sc_scatter_s1(w1e, row, jnp.zeros((N, 16), jnp.float32))
    s1r = _sc_gather_s1(s1, row)
    payload = _tc_payload(hcT, cd16, edge_attr, alpha, s1r,
                          Wv_r, Wv_h, Wv_e, bv, W1, W2p)
    partials = _sc_scatter(payload, row, jnp.zeros((N, TW), jnp.float32))
    h_out, coord_out9 = _tc_finalize(partials, h, coord9)
    return (h_out, coord_out9.reshape(N, C, 3))


# speedup vs baseline: 11.7163x; 1.1337x over previous
import functools

import jax
import jax.numpy as jnp
from jax import lax
from jax.experimental import pallas as pl
from jax.experimental.pallas import tpu as pltpu
from jax.experimental.pallas import tpu_sc as plsc

N = 10000
E = 320000
D = 128
C = 3
ED = 16
H = 128

TW = 144
EB = 2560
NBLK = E // EB
LAM = 8.0

NC = 2
NS = 16
NW = NC * NS
EPW = E // NW
EPS = E // NS
CH = 80

SL = 624
SLAST = N - SL * (NS - 1)


def _sc_mesh():
    return plsc.VectorSubcoreMesh(
        core_axis_name="c", subcore_axis_name="s",
        num_cores=NC, num_subcores=NS)


def _sc_gather(table, row, col):
    @functools.partial(
        pl.kernel, mesh=_sc_mesh(),
        out_type=(jax.ShapeDtypeStruct((E, TW), jnp.float32),
                  jax.ShapeDtypeStruct((E, TW), jnp.float32)),
        scratch_types=[
            pltpu.VMEM((EPW,), jnp.int32),
            pltpu.VMEM((EPW,), jnp.int32),
            pltpu.VMEM((CH, TW), jnp.float32),
            pltpu.VMEM((CH, TW), jnp.float32),
            pltpu.VMEM((CH, TW), jnp.float32),
            pltpu.VMEM((CH, TW), jnp.float32),
            pltpu.SemaphoreType.DMA,
            pltpu.SemaphoreType.DMA,
            pltpu.SemaphoreType.DMA,
        ],
        compiler_params=pltpu.CompilerParams(use_tc_tiling_on_sc=False),
    )
    def gather_kernel(t_hbm, row_hbm, col_hbm, outr_hbm, outc_hbm,
                      idxr_all, idxc_all, br0, bc0, br1, bc1,
                      semg0, semg1, semw):
        wid = lax.axis_index("s") * NC + lax.axis_index("c")
        base = wid * EPW
        pltpu.sync_copy(row_hbm.at[pl.ds(base, EPW)], idxr_all)
        pltpu.sync_copy(col_hbm.at[pl.ds(base, EPW)], idxc_all)

        def g_start(ch, br, bc, sem):
            s = pl.ds(ch * CH, CH)
            a = pltpu.async_copy(t_hbm.at[idxr_all.at[s]], br, sem)
            b = pltpu.async_copy(t_hbm.at[idxc_all.at[s]], bc, sem)
            return a, b

        def w_start(ch, br, bc, sem):
            off = base + ch * CH
            a = pltpu.async_copy(br, outr_hbm.at[pl.ds(off, CH)], sem)
            b = pltpu.async_copy(bc, outc_hbm.at[pl.ds(off, CH)], sem)
            return a, b

        def body(j, carry):
            c0 = 2 * j
            g0a, g0b = g_start(c0, br0, bc0, semg0)
            g1a, g1b = g_start(c0 + 1, br1, bc1, semg1)
            g0a.wait()
            g0b.wait()
            w0a, w0b = w_start(c0, br0, bc0, semw)
            g1a.wait()
            g1b.wait()
            w1a, w1b = w_start(c0 + 1, br1, bc1, semw)
            w0a.wait()
            w0b.wait()
            w1a.wait()
            w1b.wait()
            return carry

        npair = (EPW // CH) // 2
        lax.fori_loop(0, npair, body, 0)
        tc = EPW // CH - 1
        ga, gb = g_start(tc, br0, bc0, semg0)
        ga.wait()
        gb.wait()
        wa, wb = w_start(tc, br0, bc0, semw)
        wa.wait()
        wb.wait()

    return gather_kernel(table, row, col)


def _sc_gather_s1(s1, row):
    @functools.partial(
        pl.kernel, mesh=_sc_mesh(),
        out_type=jax.ShapeDtypeStruct((E, 16), jnp.float32),
        scratch_types=[
            pltpu.VMEM((EPW,), jnp.int32),
            pltpu.VMEM((CH, 16), jnp.float32),
            pltpu.VMEM((CH, 16), jnp.float32),
            pltpu.SemaphoreType.DMA,
            pltpu.SemaphoreType.DMA,
            pltpu.SemaphoreType.DMA,
        ],
        compiler_params=pltpu.CompilerParams(use_tc_tiling_on_sc=False),
    )
    def gather_kernel(s1_hbm, row_hbm, out_hbm, idx_all, b0, b1,
                      semg0, semg1, semw):
        wid = lax.axis_index("s") * NC + lax.axis_index("c")
        base = wid * EPW
        pltpu.sync_copy(row_hbm.at[pl.ds(base, EPW)], idx_all)

        def g_start(ch, b, sem):
            return pltpu.async_copy(
                s1_hbm.at[idx_all.at[pl.ds(ch * CH, CH)]], b, sem)

        def w_start(ch, b, sem):
            off = base + ch * CH
            return pltpu.async_copy(b, out_hbm.at[pl.ds(off, CH)], sem)

        def body(j, carry):
            c0 = 2 * j
            g0 = g_start(c0, b0, semg0)
            g1 = g_start(c0 + 1, b1, semg1)
            g0.wait()
            w0 = w_start(c0, b0, semw)
            g1.wait()
            w1 = w_start(c0 + 1, b1, semw)
            w0.wait()
            w1.wait()
            return carry

        lax.fori_loop(0, (EPW // CH) // 2, body, 0)
        tc = EPW // CH - 1
        g = g_start(tc, b0, semg0)
        g.wait()
        w = w_start(tc, b0, semw)
        w.wait()

    return gather_kernel(s1, row)


def _sc_scatter_s1(w1e, row, zeros_n16):
    @functools.partial(
        pl.kernel, mesh=_sc_mesh(),
        out_type=jax.ShapeDtypeStruct((N, 16), jnp.float32),
        scratch_types=[
            pltpu.VMEM((CH,), jnp.int32),
            pltpu.VMEM((CH,), jnp.int32),
            pltpu.VMEM((CH, 16), jnp.float32),
            pltpu.VMEM((CH, 16), jnp.float32),
            pltpu.VMEM_SHARED((N, 16), jnp.float32),
            pltpu.SemaphoreType.DMA,
            pltpu.SemaphoreType.DMA,
        ],
        compiler_params=pltpu.CompilerParams(use_tc_tiling_on_sc=False),
    )
    def scatter_kernel(w_hbm, row_hbm, z_hbm, out_hbm, idx0_v, idx1_v,
                       buf0_v, buf1_v, acc_sh, sem0, sem1):
        cid = lax.axis_index("c")
        sid = lax.axis_index("s")

        @pl.when(cid == 0)
        def _():
            @pl.when(sid < NS - 1)
            def _():
                pltpu.sync_copy(z_hbm.at[pl.ds(sid * SL, SL)],
                                acc_sh.at[pl.ds(sid * SL, SL)])

            @pl.when(sid == NS - 1)
            def _():
                pltpu.sync_copy(z_hbm.at[pl.ds((NS - 1) * SL, SLAST)],
                                acc_sh.at[pl.ds((NS - 1) * SL, SLAST)])

            plsc.subcore_barrier()
            base = sid * EPS

            def l_start(ch, idx_v, buf_v, sem):
                off = base + ch * CH
                a = pltpu.async_copy(row_hbm.at[pl.ds(off, CH)], idx_v, sem)
                b = pltpu.async_copy(w_hbm.at[pl.ds(off, CH)], buf_v, sem)
                return a, b

            def body(j, carry):
                c0 = 2 * j
                a0, b0 = l_start(c0, idx0_v, buf0_v, sem0)
                a1, b1 = l_start(c0 + 1, idx1_v, buf1_v, sem1)
                a0.wait()
                b0.wait()
                pltpu.sync_copy(buf0_v, acc_sh.at[idx0_v], add=True)
                a1.wait()
                b1.wait()
                pltpu.sync_copy(buf1_v, acc_sh.at[idx1_v], add=True)
                return carry

            lax.fori_loop(0, (EPS // CH) // 2, body, 0)
            plsc.subcore_barrier()

            @pl.when(sid < NS - 1)
            def _():
                pltpu.sync_copy(acc_sh.at[pl.ds(sid * SL, SL)],
                                out_hbm.at[pl.ds(sid * SL, SL)])

            @pl.when(sid == NS - 1)
            def _():
                pltpu.sync_copy(acc_sh.at[pl.ds((NS - 1) * SL, SLAST)],
                                out_hbm.at[pl.ds((NS - 1) * SL, SLAST)])

    return scatter_kernel(w1e, row, zeros_n16)


def _sc_scatter(payload, row, zeros_nt):
    @functools.partial(
        pl.kernel, mesh=_sc_mesh(),
        out_type=jax.ShapeDtypeStruct((NC, N, TW), jnp.float32),
        scratch_types=[
            pltpu.VMEM((CH,), jnp.int32),
            pltpu.VMEM((CH,), jnp.int32),
            pltpu.VMEM((CH, TW), jnp.float32),
            pltpu.VMEM((CH, TW), jnp.float32),
            pltpu.VMEM_SHARED((N, TW), jnp.float32),
            pltpu.SemaphoreType.DMA,
            pltpu.SemaphoreType.DMA,
        ],
        compiler_params=pltpu.CompilerParams(use_tc_tiling_on_sc=False),
    )
    def scatter_kernel(pay_hbm, row_hbm, z_hbm, out_hbm, idx0_v, idx1_v,
                       buf0_v, buf1_v, acc_sh, sem0, sem1):
        cid = lax.axis_index("c")
        sid = lax.axis_index("s")
        wid = sid * NC + cid
        base = wid * EPW

        @pl.when(sid < NS - 1)
        def _():
            pltpu.sync_copy(z_hbm.at[pl.ds(sid * SL, SL)],
                            acc_sh.at[pl.ds(sid * SL, SL)])

        @pl.when(sid == NS - 1)
        def _():
            pltpu.sync_copy(z_hbm.at[pl.ds((NS - 1) * SL, SLAST)],
                            acc_sh.at[pl.ds((NS - 1) * SL, SLAST)])

        plsc.subcore_barrier()

        def l_start(ch, idx_v, buf_v, sem):
            off = base + ch * CH
            a = pltpu.async_copy(row_hbm.at[pl.ds(off, CH)], idx_v, sem)
            b = pltpu.async_copy(pay_hbm.at[pl.ds(off, CH)], buf_v, sem)
            return a, b

        def body(j, carry):
            c0 = 2 * j
            a0, b0 = l_start(c0, idx0_v, buf0_v, sem0)
            a1, b1 = l_start(c0 + 1, idx1_v, buf1_v, sem1)
            a0.wait()
            b0.wait()
            pltpu.sync_copy(buf0_v, acc_sh.at[idx0_v], add=True)
            a1.wait()
            b1.wait()
            pltpu.sync_copy(buf1_v, acc_sh.at[idx1_v], add=True)
            return carry

        npair = (EPW // CH) // 2
        lax.fori_loop(0, npair, body, 0)
        ta, tb = l_start(EPW // CH - 1, idx0_v, buf0_v, sem0)
        ta.wait()
        tb.wait()
        pltpu.sync_copy(buf0_v, acc_sh.at[idx0_v], add=True)
        plsc.subcore_barrier()

        @pl.when(sid < NS - 1)
        def _():
            pltpu.sync_copy(acc_sh.at[pl.ds(sid * SL, SL)],
                            out_hbm.at[cid, pl.ds(sid * SL, SL)])

        @pl.when(sid == NS - 1)
        def _():
            pltpu.sync_copy(acc_sh.at[pl.ds((NS - 1) * SL, SLAST)],
                            out_hbm.at[cid, pl.ds((NS - 1) * SL, SLAST)])

    return scatter_kernel(payload, row, zeros_nt)


def _radial16(cd):
    cols = []
    for i in range(C):
        a = cd[:, 3 * i:3 * i + 3]
        for j in range(C):
            b = cd[:, 3 * j:3 * j + 3]
            cols.append(jnp.sum(a * b, axis=1, keepdims=True))
    cols.append(jnp.zeros((cd.shape[0], 7), jnp.float32))
    return jnp.concatenate(cols, axis=1)


def _tc_alpha(hrT, hcT, ea, Wq, bq, Wk_r, Wk_h, Wk_e, bk):
    def body(hr_ref, hc_ref, ea_ref, wq_ref, bq_ref, wkr_ref, wkh_ref,
             wke_ref, bk_ref, alpha_ref, cd_ref, w1_ref):
        hr = hr_ref[:, :D]
        cr = hr_ref[:, D:D + 9]
        hc = hc_ref[:, :D]
        cc = hc_ref[:, D:D + 9]
        cd = cr - cc
        cd_ref[...] = jnp.concatenate(
            [cd, jnp.zeros((EB, 7), jnp.float32)], axis=1)
        rad = _radial16(cd)
        q = jnp.dot(hr, wq_ref[...], preferred_element_type=jnp.float32)
        q = q + bq_ref[...]
        k = (jnp.dot(rad, wkr_ref[...], preferred_element_type=jnp.float32)
             + jnp.dot(hc, wkh_ref[...], preferred_element_type=jnp.float32)
             + jnp.dot(ea_ref[...], wke_ref[...],
                       preferred_element_type=jnp.float32)
             + bk_ref[...])
        alpha = jnp.sum(q * k, axis=1, keepdims=True)
        alpha_ref[...] = alpha
        w1_ref[...] = jnp.concatenate(
            [jnp.exp(alpha * (1.0 / LAM)), jnp.zeros((EB, 15), jnp.float32)],
            axis=1)

    full = lambda s: pl.BlockSpec(s, lambda i: (0, 0))
    return pl.pallas_call(
        body,
        grid=(NBLK,),
        in_specs=[
            pl.BlockSpec((EB, TW), lambda i: (i, 0)),
            pl.BlockSpec((EB, TW), lambda i: (i, 0)),
            pl.BlockSpec((EB, ED), lambda i: (i, 0)),
            full((D, H)), full((1, H)), full((16, H)), full((D, H)),
            full((ED, H)), full((1, H)),
        ],
        out_specs=[
            pl.BlockSpec((EB, 1), lambda i: (i, 0)),
            pl.BlockSpec((EB, 16), lambda i: (i, 0)),
            pl.BlockSpec((EB, 16), lambda i: (i, 0)),
        ],
        out_shape=[
            jax.ShapeDtypeStruct((E, 1), jnp.float32),
            jax.ShapeDtypeStruct((E, 16), jnp.float32),
            jax.ShapeDtypeStruct((E, 16), jnp.float32),
        ],
        compiler_params=pltpu.CompilerParams(
            dimension_semantics=("parallel",)),
    )(hrT, hcT, ea, Wq, bq, Wk_r, Wk_h, Wk_e, bk)


def _tc_payload(hcT, cd16, ea, alpha, s1r, Wv_r, Wv_h, Wv_e, bv, W1, W2p):
    def body(hc_ref, cd_ref, ea_ref, al_ref, s1_ref, wvr_ref, wvh_ref,
             wve_ref, bv_ref, w1_ref, w2_ref, out_ref):
        hc = hc_ref[:, :D]
        cd = cd_ref[:, :9]
        rad = _radial16(cd)
        v = (jnp.dot(rad, wvr_ref[...], preferred_element_type=jnp.float32)
             + jnp.dot(hc, wvh_ref[...], preferred_element_type=jnp.float32)
             + jnp.dot(ea_ref[...], wve_ref[...],
                       preferred_element_type=jnp.float32)
             + bv_ref[...])
        ex = jnp.exp(al_ref[...] - LAM * jnp.log(s1_ref[:, 0:1]))
        u = jnp.dot(v, w1_ref[...], preferred_element_type=jnp.float32)
        u = u * jax.nn.sigmoid(u)
        cv = jnp.dot(u, w2_ref[...], preferred_element_type=jnp.float32)
        tcols = []
        for i in range(C):
            ci = cv[:, i:i + 1]
            for d in range(3):
                tcols.append(cd[:, 3 * i + d:3 * i + d + 1] * ci)
        trans = jnp.concatenate(tcols, axis=1)
        out_ref[...] = jnp.concatenate(
            [ex * v, ex * trans, ex, jnp.zeros((EB, 6), jnp.float32)],
            axis=1)

    full = lambda s: pl.BlockSpec(s, lambda i: (0, 0))
    return pl.pallas_call(
        body,
        grid=(NBLK,),
        in_specs=[
            pl.BlockSpec((EB, TW), lambda i: (i, 0)),
            pl.BlockSpec((EB, 16), lambda i: (i, 0)),
            pl.BlockSpec((EB, ED), lambda i: (i, 0)),
            pl.BlockSpec((EB, 1), lambda i: (i, 0)),
            pl.BlockSpec((EB, 16), lambda i: (i, 0)),
            full((16, H)), full((D, H)), full((ED, H)), full((1, H)),
            full((H, 2 * H)), full((2 * H, 16)),
        ],
        out_specs=pl.BlockSpec((EB, TW), lambda i: (i, 0)),
        out_shape=jax.ShapeDtypeStruct((E, TW), jnp.float32),
        compiler_params=pltpu.CompilerParams(
            dimension_semantics=("parallel",)),
    )(hcT, cd16, ea, alpha, s1r, Wv_r, Wv_h, Wv_e, bv, W1, W2p)


def _tc_finalize(partials, h, coord9):
    NB = 1000

    def body(p_ref, h_ref, c_ref, ho_ref, co_ref):
        u = p_ref[0] + p_ref[1]
        s = u[:, D + 9:D + 10]
        inv = jnp.where(s > 0, 1.0 / jnp.where(s > 0, s, 1.0), 0.0)
        ho_ref[...] = h_ref[...] + u[:, :D] * inv
        aggc = u[:, D:D + 9] * inv
        co_ref[...] = c_ref[...] + jnp.clip(aggc, -10.0, 10.0)

    return pl.pallas_call(
        body,
        grid=(N // NB,),
        in_specs=[
            pl.BlockSpec((NC, NB, TW), lambda i: (0, i, 0)),
            pl.BlockSpec((NB, D), lambda i: (i, 0)),
            pl.BlockSpec((NB, 9), lambda i: (i, 0)),
        ],
        out_specs=[
            pl.BlockSpec((NB, D), lambda i: (i, 0)),
            pl.BlockSpec((NB, 9), lambda i: (i, 0)),
        ],
        out_shape=[
            jax.ShapeDtypeStruct((N, D), jnp.float32),
            jax.ShapeDtypeStruct((N, 9), jnp.float32),
        ],
        compiler_params=pltpu.CompilerParams(
            dimension_semantics=("parallel",)),
    )(partials, h, coord9)


def kernel(h, edge_index, coord, edge_attr, Wq, bq, Wkv, bkv, W1, W2):
    row = edge_index[0]
    col = edge_index[1]
    coord9 = coord.reshape(N, C * 3)

    table = jnp.concatenate(
        [h, coord9, jnp.zeros((N, TW - D - C * 3), jnp.float32)], axis=1)

    Wk = Wkv[:, 0::2]
    Wv = Wkv[:, 1::2]
    bk = bkv[0::2].reshape(1, H)
    bv = bkv[1::2].reshape(1, H)
    Wk_r = jnp.concatenate([Wk[:C * C], jnp.zeros((7, H), jnp.float32)], 0)
    Wv_r = jnp.concatenate([Wv[:C * C], jnp.zeros((7, H), jnp.float32)], 0)
    Wk_h = Wk[C * C:C * C + D]
    Wv_h = Wv[C * C:C * C + D]
    Wk_e = Wk[C * C + D:]
    Wv_e = Wv[C * C + D:]
    W2p = jnp.concatenate([W2, jnp.zeros((2 * H, 16 - C), jnp.float32)], 1)
    bq2 = bq.reshape(1, H)

    hrT, hcT = _sc_gather(table, row, col)
    alpha, cd16, w1e = _tc_alpha(hrT, hcT, edge_attr,
                                 Wq, bq2, Wk_r, Wk_h, Wk_e, bk)
    s1 = _sc_scatter_s1(w1e, row, jnp.zeros((N, 16), jnp.float32))
    s1r = _sc_gather_s1(s1, row)
    payload = _tc_payload(hcT, cd16, edge_attr, alpha, s1r,
                          Wv_r, Wv_h, Wv_e, bv, W1, W2p)
    partials = _sc_scatter(payload, row, jnp.zeros((N, TW), jnp.float32))
    h_out, coord_out9 = _tc_finalize(partials, h, coord9)
    return (h_out, coord_out9.reshape(N, C, 3))
